# baseline (speedup 1.0000x reference)
import jax
import jax.numpy as jnp
from jax.experimental import pallas as pl
from jax.experimental.pallas import tpu as pltpu

_K = 32
_LAMBDA = 0.1
_TEMP = 0.1
_NEG = -3.0e38
_R = 256


def _phase1_body(f_ref, fb_ref, sim_ref, a_ref):
    f = f_ref[...]
    B = f.shape[0]
    n = jnp.sqrt(jnp.sum(f * f, axis=1, keepdims=True))
    x = f / jnp.maximum(n, 1e-12)
    fb = fb_ref[...]
    nb = jnp.sqrt(jnp.sum(fb * fb, axis=1, keepdims=True))
    xb = fb / jnp.maximum(nb, 1e-12)
    sim = jax.lax.dot_general(
        xb, x, (((1,), (1,)), ((), ())), preferred_element_type=jnp.float32)
    sim_ref[...] = sim
    iota = jax.lax.broadcasted_iota(jnp.int32, (_R, B), 1)

    bits = jax.lax.bitcast_convert_type(sim, jnp.uint32)
    ukey = jnp.where(
        (bits >> 31) != 0, ~bits, bits | jnp.uint32(0x80000000))

    kk = _K + 1
    hi = (ukey >> 16).astype(jnp.int32)
    lo = (ukey & jnp.uint32(0xFFFF)).astype(jnp.int32)
    his = (hi - 32768).astype(jnp.int16)
    los = (lo - 32768).astype(jnp.int16)

    def hi_round(t, p):
        cand = p | (jnp.int32(1) << (jnp.int32(15) - t))
        cands = (cand - 32768).astype(jnp.int16)
        cnt = jnp.sum((his >= cands).astype(jnp.int16), axis=1,
                      keepdims=True).astype(jnp.int32)
        return jnp.where(cnt >= kk, cand, p)

    hi_t = jax.lax.fori_loop(0, 16, hi_round, jnp.zeros((_R, 1), jnp.int32))
    hi_ts = (hi_t - 32768).astype(jnp.int16)
    himatch = his == hi_ts
    cnt_hi_gt = jnp.sum((his > hi_ts).astype(jnp.int16), axis=1,
                        keepdims=True).astype(jnp.int32)
    need_lo = kk - cnt_hi_gt
    los_m = jnp.where(himatch, los, jnp.int16(-32768))

    def lo_round(t, p):
        cand = p | (jnp.int32(1) << (jnp.int32(15) - t))
        cands = (cand - 32768).astype(jnp.int16)
        cnt = jnp.sum((los_m >= cands).astype(jnp.int16), axis=1,
                      keepdims=True).astype(jnp.int32)
        return jnp.where(cnt >= need_lo, cand, p)

    lo_t = jax.lax.fori_loop(0, 16, lo_round, jnp.zeros((_R, 1), jnp.int32))
    lo_ts = (lo_t - 32768).astype(jnp.int16)

    t33 = ((hi_t.astype(jnp.uint32) << 16)
           | lo_t.astype(jnp.uint32))
    eq = ukey == t33
    gt16 = ((his > hi_ts).astype(jnp.int16)
            + (los_m > lo_ts).astype(jnp.int16))
    gt_f = gt16.astype(jnp.float32)
    cnt_gt = cnt_hi_gt + jnp.sum(
        (los_m > lo_ts).astype(jnp.int16), axis=1,
        keepdims=True).astype(jnp.int32)
    cnt_eq = jnp.sum(eq.astype(jnp.int32), axis=1, keepdims=True)
    need = kk - cnt_gt

    def no_tie():
        return gt_f + jnp.where(eq, 1.0, 0.0)

    def tie_break():
        w = jnp.where(eq, B - iota, 0)

        def idx_round(t, prefix):
            cand = prefix | (jnp.int32(1) << (jnp.int32(12) - t))
            cnt = jnp.sum((w >= cand).astype(jnp.int32), axis=1,
                          keepdims=True)
            return jnp.where(cnt >= need, cand, prefix)

        wstar = jax.lax.fori_loop(
            0, 13, idx_round, jnp.zeros((_R, 1), jnp.int32))
        return gt_f + jnp.where(w >= wstar, 1.0, 0.0)

    sel = jax.lax.cond(jnp.all(cnt_eq == need), no_tie, tie_break)

    m = jnp.max(sim, axis=1, keepdims=True)
    i0 = jnp.min(jnp.where(sim == m, iota, B), axis=1, keepdims=True)
    a_ref[...] = jnp.where(iota == i0, 0.0, sel)


def _spmm_body(a_row_ref, a_col_ref, v_ref, out_ref):
    v = v_ref[...]
    acc = jax.lax.dot_general(
        a_row_ref[...], v, (((1,), (0,)), ((), ())),
        preferred_element_type=jnp.float32)
    acc = acc + jax.lax.dot_general(
        a_col_ref[...], v, (((0,), (0,)), ((), ())),
        preferred_element_type=jnp.float32)
    out_ref[...] = acc * (1.0 / _K)


def _phase3_body(sim_ref, fb_ref, f_ref, g_ref, out_ref):
    f = f_ref[...]
    g = g_ref[...]
    fb = fb_ref[...]
    gn = g / jnp.maximum(
        jnp.sqrt(jnp.sum(g * g, axis=1, keepdims=True)), 1e-8)
    fnb = fb / jnp.maximum(
        jnp.sqrt(jnp.sum(fb * fb, axis=1, keepdims=True)), 1e-8)
    cos = jax.lax.dot_general(
        fnb, gn, (((1,), (1,)), ((), ())), preferred_element_type=jnp.float32)
    logits = (sim_ref[...] + _LAMBDA * cos) / _TEMP
    m = jnp.max(logits, axis=1, keepdims=True)
    e = jnp.exp(logits - m)
    s = jnp.sum(e, axis=1, keepdims=True)
    acc = jax.lax.dot_general(
        e, f, (((1,), (0,)), ((), ())), preferred_element_type=jnp.float32)
    out_ref[...] = acc / s


def kernel(features):
    B, D = features.shape
    nblk = B // _R
    f32 = jnp.float32

    sim, a = pl.pallas_call(
        _phase1_body,
        grid=(nblk,),
        in_specs=[
            pl.BlockSpec((B, D), lambda i: (0, 0)),
            pl.BlockSpec((_R, D), lambda i: (i, 0)),
        ],
        out_specs=[
            pl.BlockSpec((_R, B), lambda i: (i, 0)),
            pl.BlockSpec((_R, B), lambda i: (i, 0)),
        ],
        out_shape=[
            jax.ShapeDtypeStruct((B, B), f32),
            jax.ShapeDtypeStruct((B, B), f32),
        ],
    )(features, features)

    def spmm(v):
        return pl.pallas_call(
            _spmm_body,
            grid=(nblk,),
            in_specs=[
                pl.BlockSpec((_R, B), lambda i: (i, 0)),
                pl.BlockSpec((B, _R), lambda i: (0, i)),
                pl.BlockSpec((B, D), lambda i: (0, 0)),
            ],
            out_specs=pl.BlockSpec((_R, D), lambda i: (i, 0)),
            out_shape=jax.ShapeDtypeStruct((B, D), f32),
        )(a, a, v)

    diff1 = spmm(features)
    geo = spmm(diff1)

    enhanced = pl.pallas_call(
        _phase3_body,
        grid=(nblk,),
        in_specs=[
            pl.BlockSpec((_R, B), lambda i: (i, 0)),
            pl.BlockSpec((_R, D), lambda i: (i, 0)),
            pl.BlockSpec((B, D), lambda i: (0, 0)),
            pl.BlockSpec((B, D), lambda i: (0, 0)),
        ],
        out_specs=pl.BlockSpec((_R, D), lambda i: (i, 0)),
        out_shape=jax.ShapeDtypeStruct((B, D), f32),
    )(sim, features, features, geo)

    return enhanced


# speedup vs baseline: 1.5672x; 1.5672x over previous
import jax
import jax.numpy as jnp
from jax.experimental import pallas as pl
from jax.experimental.pallas import tpu as pltpu

_K = 32
_LAMBDA = 0.1
_TEMP = 0.1
_NEG = -3.0e38
_R = 256


def _phase1_body(f_ref, fb_ref, sim_ref, a_ref):
    f = f_ref[...]
    B = f.shape[0]
    n = jnp.sqrt(jnp.sum(f * f, axis=1, keepdims=True))
    x = f / jnp.maximum(n, 1e-12)
    fb = fb_ref[...]
    nb = jnp.sqrt(jnp.sum(fb * fb, axis=1, keepdims=True))
    xb = fb / jnp.maximum(nb, 1e-12)
    sim = jax.lax.dot_general(
        xb, x, (((1,), (1,)), ((), ())), preferred_element_type=jnp.float32)
    sim_ref[...] = sim
    iota = jax.lax.broadcasted_iota(jnp.int32, (_R, B), 1)

    bits = jax.lax.bitcast_convert_type(sim, jnp.uint32)
    ukey = jnp.where(
        (bits >> 31) != 0, ~bits, bits | jnp.uint32(0x80000000))

    kk = _K + 1

    def bit_round(t, prefix):
        cand = prefix | (jnp.uint32(1) << (jnp.uint32(31) - t.astype(jnp.uint32)))
        cnt = jnp.sum((ukey >= cand).astype(jnp.int32), axis=1, keepdims=True)
        return jnp.where(cnt >= kk, cand, prefix)

    t33 = jax.lax.fori_loop(
        0, 32, bit_round, jnp.zeros((_R, 1), jnp.uint32))

    eq = ukey == t33
    gt_f = jnp.where(ukey > t33, 1.0, 0.0)
    cnt_gt = jnp.sum(gt_f.astype(jnp.int32), axis=1, keepdims=True)
    cnt_eq = jnp.sum(eq.astype(jnp.int32), axis=1, keepdims=True)
    need = kk - cnt_gt

    def no_tie():
        return gt_f + jnp.where(eq, 1.0, 0.0)

    def tie_break():
        w = jnp.where(eq, B - iota, 0)

        def idx_round(t, prefix):
            cand = prefix | (jnp.int32(1) << (jnp.int32(12) - t))
            cnt = jnp.sum((w >= cand).astype(jnp.int32), axis=1,
                          keepdims=True)
            return jnp.where(cnt >= need, cand, prefix)

        wstar = jax.lax.fori_loop(
            0, 13, idx_round, jnp.zeros((_R, 1), jnp.int32))
        return gt_f + jnp.where(w >= wstar, 1.0, 0.0)

    sel = jax.lax.cond(jnp.all(cnt_eq == need), no_tie, tie_break)

    m = jnp.max(sim, axis=1, keepdims=True)
    i0 = jnp.min(jnp.where(sim == m, iota, B), axis=1, keepdims=True)
    a_ref[...] = jnp.where(iota == i0, 0.0, sel)


def _spmm_body(a_row_ref, a_col_ref, v_ref, out_ref):
    v = v_ref[...]
    acc = jax.lax.dot_general(
        a_row_ref[...], v, (((1,), (0,)), ((), ())),
        preferred_element_type=jnp.float32)
    acc = acc + jax.lax.dot_general(
        a_col_ref[...], v, (((0,), (0,)), ((), ())),
        preferred_element_type=jnp.float32)
    out_ref[...] = acc * (1.0 / _K)


def _phase3_body(sim_ref, fb_ref, f_ref, g_ref, out_ref):
    f = f_ref[...]
    g = g_ref[...]
    fb = fb_ref[...]
    gn = g / jnp.maximum(
        jnp.sqrt(jnp.sum(g * g, axis=1, keepdims=True)), 1e-8)
    fnb = fb / jnp.maximum(
        jnp.sqrt(jnp.sum(fb * fb, axis=1, keepdims=True)), 1e-8)
    cos = jax.lax.dot_general(
        fnb, gn, (((1,), (1,)), ((), ())), preferred_element_type=jnp.float32)
    logits = (sim_ref[...] + _LAMBDA * cos) / _TEMP
    m = jnp.max(logits, axis=1, keepdims=True)
    e = jnp.exp(logits - m)
    s = jnp.sum(e, axis=1, keepdims=True)
    acc = jax.lax.dot_general(
        e, f, (((1,), (0,)), ((), ())), preferred_element_type=jnp.float32)
    out_ref[...] = acc / s


def kernel(features):
    B, D = features.shape
    nblk = B // _R
    f32 = jnp.float32

    sim, a = pl.pallas_call(
        _phase1_body,
        grid=(nblk,),
        in_specs=[
            pl.BlockSpec((B, D), lambda i: (0, 0)),
            pl.BlockSpec((_R, D), lambda i: (i, 0)),
        ],
        out_specs=[
            pl.BlockSpec((_R, B), lambda i: (i, 0)),
            pl.BlockSpec((_R, B), lambda i: (i, 0)),
        ],
        out_shape=[
            jax.ShapeDtypeStruct((B, B), f32),
            jax.ShapeDtypeStruct((B, B), f32),
        ],
    )(features, features)

    def spmm(v):
        return pl.pallas_call(
            _spmm_body,
            grid=(nblk,),
            in_specs=[
                pl.BlockSpec((_R, B), lambda i: (i, 0)),
                pl.BlockSpec((B, _R), lambda i: (0, i)),
                pl.BlockSpec((B, D), lambda i: (0, 0)),
            ],
            out_specs=pl.BlockSpec((_R, D), lambda i: (i, 0)),
            out_shape=jax.ShapeDtypeStruct((B, D), f32),
        )(a, a, v)

    diff1 = spmm(features)
    geo = spmm(diff1)

    enhanced = pl.pallas_call(
        _phase3_body,
        grid=(nblk,),
        in_specs=[
            pl.BlockSpec((_R, B), lambda i: (i, 0)),
            pl.BlockSpec((_R, D), lambda i: (i, 0)),
            pl.BlockSpec((B, D), lambda i: (0, 0)),
            pl.BlockSpec((B, D), lambda i: (0, 0)),
        ],
        out_specs=pl.BlockSpec((_R, D), lambda i: (i, 0)),
        out_shape=jax.ShapeDtypeStruct((B, D), f32),
    )(sim, features, features, geo)

    return enhanced


# speedup vs baseline: 1.6372x; 1.0447x over previous
import jax
import jax.numpy as jnp
from jax.experimental import pallas as pl
from jax.experimental.pallas import tpu as pltpu

_K = 32
_LAMBDA = 0.1
_TEMP = 0.1
_R = 256


def _phase1_body(f_ref, fb_ref, sim_ref, a_ref, d1g_ref, d1s_ref):
    i = pl.program_id(0)
    f = f_ref[...]
    B = f.shape[0]
    n = jnp.sqrt(jnp.sum(f * f, axis=1, keepdims=True))
    x = f / jnp.maximum(n, 1e-12)
    fb = fb_ref[...]
    nb = jnp.sqrt(jnp.sum(fb * fb, axis=1, keepdims=True))
    xb = fb / jnp.maximum(nb, 1e-12)
    sim = jax.lax.dot_general(
        xb, x, (((1,), (1,)), ((), ())), preferred_element_type=jnp.float32)
    sim_ref[...] = sim
    iota = jax.lax.broadcasted_iota(jnp.int32, (_R, B), 1)

    bits = jax.lax.bitcast_convert_type(sim, jnp.uint32)
    ukey = jnp.where(
        (bits >> 31) != 0, ~bits, bits | jnp.uint32(0x80000000))

    kk = _K + 1

    def bit_round(t, prefix):
        cand = prefix | (jnp.uint32(1) << (jnp.uint32(31) - t.astype(jnp.uint32)))
        cnt = jnp.sum((ukey >= cand).astype(jnp.int32), axis=1, keepdims=True)
        return jnp.where(cnt >= kk, cand, prefix)

    t33 = jax.lax.fori_loop(
        0, 32, bit_round, jnp.zeros((_R, 1), jnp.uint32))

    eq = ukey == t33
    gt_f = jnp.where(ukey > t33, 1.0, 0.0)
    cnt_gt = jnp.sum(gt_f.astype(jnp.int32), axis=1, keepdims=True)
    cnt_eq = jnp.sum(eq.astype(jnp.int32), axis=1, keepdims=True)
    need = kk - cnt_gt

    def no_tie():
        return gt_f + jnp.where(eq, 1.0, 0.0)

    def tie_break():
        w = jnp.where(eq, B - iota, 0)

        def idx_round(t, prefix):
            cand = prefix | (jnp.int32(1) << (jnp.int32(12) - t))
            cnt = jnp.sum((w >= cand).astype(jnp.int32), axis=1,
                          keepdims=True)
            return jnp.where(cnt >= need, cand, prefix)

        wstar = jax.lax.fori_loop(
            0, 13, idx_round, jnp.zeros((_R, 1), jnp.int32))
        return gt_f + jnp.where(w >= wstar, 1.0, 0.0)

    sel = jax.lax.cond(jnp.all(cnt_eq == need), no_tie, tie_break)

    m = jnp.max(sim, axis=1, keepdims=True)
    i0 = jnp.min(jnp.where(sim == m, iota, B), axis=1, keepdims=True)
    af = jnp.where(iota == i0, 0.0, sel)
    a_ref[...] = af.astype(jnp.bfloat16)

    d1g_ref[...] = jax.lax.dot_general(
        af, f, (((1,), (0,)), ((), ())), preferred_element_type=jnp.float32)
    contrib = jax.lax.dot_general(
        af, fb, (((0,), (0,)), ((), ())), preferred_element_type=jnp.float32)

    @pl.when(i == 0)
    def _():
        d1s_ref[...] = contrib

    @pl.when(i > 0)
    def _():
        d1s_ref[...] += contrib


def _phase2_body(a_ref, d1g_ref, d1s_ref, d1gb_ref, d1sb_ref,
                 gg_ref, gs_ref):
    i = pl.program_id(0)
    a16 = a_ref[...]
    v = (d1g_ref[...] + d1s_ref[...]) * (1.0 / _K)
    v16 = v.astype(jnp.bfloat16)
    vb = (d1gb_ref[...] + d1sb_ref[...]) * (1.0 / _K)
    vb16 = vb.astype(jnp.bfloat16)
    gg_ref[...] = jax.lax.dot_general(
        a16, v16, (((1,), (0,)), ((), ())), preferred_element_type=jnp.float32)
    contrib = jax.lax.dot_general(
        a16, vb16, (((0,), (0,)), ((), ())), preferred_element_type=jnp.float32)

    @pl.when(i == 0)
    def _():
        gs_ref[...] = contrib

    @pl.when(i > 0)
    def _():
        gs_ref[...] += contrib


def _phase3_body(sim_ref, fb_ref, f_ref, gg_ref, gs_ref, out_ref):
    f = f_ref[...]
    g = (gg_ref[...] + gs_ref[...]) * (1.0 / _K)
    fb = fb_ref[...]
    gn = g / jnp.maximum(
        jnp.sqrt(jnp.sum(g * g, axis=1, keepdims=True)), 1e-8)
    fnb = fb / jnp.maximum(
        jnp.sqrt(jnp.sum(fb * fb, axis=1, keepdims=True)), 1e-8)
    cos = jax.lax.dot_general(
        fnb.astype(jnp.bfloat16), gn.astype(jnp.bfloat16),
        (((1,), (1,)), ((), ())), preferred_element_type=jnp.float32)
    logits = (sim_ref[...] + _LAMBDA * cos) / _TEMP
    m = jnp.max(logits, axis=1, keepdims=True)
    e = jnp.exp(logits - m)
    s = jnp.sum(e, axis=1, keepdims=True)
    acc = jax.lax.dot_general(
        e, f, (((1,), (0,)), ((), ())), preferred_element_type=jnp.float32)
    out_ref[...] = acc / s


def kernel(features):
    B, D = features.shape
    nblk = B // _R
    f32 = jnp.float32

    sim, a, d1g, d1s = pl.pallas_call(
        _phase1_body,
        grid=(nblk,),
        in_specs=[
            pl.BlockSpec((B, D), lambda i: (0, 0)),
            pl.BlockSpec((_R, D), lambda i: (i, 0)),
        ],
        out_specs=[
            pl.BlockSpec((_R, B), lambda i: (i, 0)),
            pl.BlockSpec((_R, B), lambda i: (i, 0)),
            pl.BlockSpec((_R, D), lambda i: (i, 0)),
            pl.BlockSpec((B, D), lambda i: (0, 0)),
        ],
        out_shape=[
            jax.ShapeDtypeStruct((B, B), f32),
            jax.ShapeDtypeStruct((B, B), jnp.bfloat16),
            jax.ShapeDtypeStruct((B, D), f32),
            jax.ShapeDtypeStruct((B, D), f32),
        ],
    )(features, features)

    gg, gs = pl.pallas_call(
        _phase2_body,
        grid=(nblk,),
        in_specs=[
            pl.BlockSpec((_R, B), lambda i: (i, 0)),
            pl.BlockSpec((B, D), lambda i: (0, 0)),
            pl.BlockSpec((B, D), lambda i: (0, 0)),
            pl.BlockSpec((_R, D), lambda i: (i, 0)),
            pl.BlockSpec((_R, D), lambda i: (i, 0)),
        ],
        out_specs=[
            pl.BlockSpec((_R, D), lambda i: (i, 0)),
            pl.BlockSpec((B, D), lambda i: (0, 0)),
        ],
        out_shape=[
            jax.ShapeDtypeStruct((B, D), f32),
            jax.ShapeDtypeStruct((B, D), f32),
        ],
    )(a, d1g, d1s, d1g, d1s)

    enhanced = pl.pallas_call(
        _phase3_body,
        grid=(nblk,),
        in_specs=[
            pl.BlockSpec((_R, B), lambda i: (i, 0)),
            pl.BlockSpec((_R, D), lambda i: (i, 0)),
            pl.BlockSpec((B, D), lambda i: (0, 0)),
            pl.BlockSpec((B, D), lambda i: (0, 0)),
            pl.BlockSpec((B, D), lambda i: (0, 0)),
        ],
        out_specs=pl.BlockSpec((_R, D), lambda i: (i, 0)),
        out_shape=jax.ShapeDtypeStruct((B, D), f32),
    )(sim, features, features, gg, gs)

    return enhanced
